# baseline (speedup 1.0000x reference)
import functools

import jax
import jax.numpy as jnp
from jax import lax
from jax.experimental import pallas as pl
from jax.experimental.pallas import tpu as pltpu
from jax.experimental.pallas import tpu_sc as plsc

F32 = jnp.float32
NC, NS, L = 2, 16, 16
NW = NC * NS
N_ENT = 10000
N_ENTP = 10240
N_RELP = 512
UP = 64
DCOL = 50
K = 80
EPT = 320000 // NW
NB = EPT // K
RB = N_ENTP // NS
ZR = 128


def _dot(x, y):
    return jnp.dot(x, y, preferred_element_type=F32)


def _one50(t):
    return t + (lax.broadcasted_iota(jnp.int32, t.shape, 1) == DCOL).astype(F32)


def _ent_prep_body(ee, wh, wt, ap, hh, tt, sh, st):
    e = ee[...]
    h = _dot(e, wh[...])
    t = _dot(e, wt[...])
    hh[...] = h
    tt[...] = _one50(t)
    sh[...] = _dot(h, ap[...])
    st[...] = _dot(t, ap[...])


def _ent_prep(ee, wh, wt, ap):
    n, d = ee.shape
    b = 2048
    full = lambda shape: pl.BlockSpec(shape, lambda i: (0, 0))
    row = lambda shape: pl.BlockSpec(shape, lambda i: (i, 0))
    return pl.pallas_call(
        _ent_prep_body,
        grid=(n // b,),
        in_specs=[row((b, d)), full((d, UP)), full((d, UP)), full((UP, 1))],
        out_specs=[row((b, UP)), row((b, UP)), row((b, 1)), row((b, 1))],
        out_shape=[jax.ShapeDtypeStruct((n, UP), F32),
                   jax.ShapeDtypeStruct((n, UP), F32),
                   jax.ShapeDtypeStruct((n, 1), F32),
                   jax.ShapeDtypeStruct((n, 1), F32)],
    )(ee, wh, wt, ap)


def _rel_prep_body(er, w0r, a0p, wr0, w1r, a1p, wr1,
                   rr0, sr0, r1, sr1, relout):
    e = er[...]
    rr = _dot(e, w0r[...])
    rr0[...] = rr
    sr0[...] = _dot(rr, a0p[...])
    rel1 = jnp.maximum(_dot(e, wr0[...]), 0.0)
    r1v = _dot(rel1, w1r[...])
    r1[...] = r1v
    sr1[...] = _dot(r1v, a1p[...])
    relout[...] = _dot(rel1, wr1[...])


def _rel_prep(erp, w0r, a0p, wr0, w1r, a1p, wr1):
    return pl.pallas_call(
        _rel_prep_body,
        out_shape=[jax.ShapeDtypeStruct((N_RELP, UP), F32),
                   jax.ShapeDtypeStruct((N_RELP, 1), F32),
                   jax.ShapeDtypeStruct((N_RELP, UP), F32),
                   jax.ShapeDtypeStruct((N_RELP, 1), F32),
                   jax.ShapeDtypeStruct((N_RELP, UP), F32)],
    )(erp, w0r, a0p, wr0, w1r, a1p, wr1)


def _norm_prep_body(p0, p1, hh, wh, wt, ap, h1, t1, sh, st):
    acc = p0[...] + p1[...]
    denom = acc[:, DCOL:DCOL + 1]
    out0 = (hh[...] * denom + acc) / (denom + 1e-16)
    h = _dot(out0, wh[...])
    t = _dot(out0, wt[...])
    h1[...] = h
    t1[...] = _one50(t)
    sh[...] = _dot(h, ap[...])
    st[...] = _dot(t, ap[...])


def _norm_prep(p0, p1, hh, wh, wt, ap):
    b = 2048
    full = lambda shape: pl.BlockSpec(shape, lambda i: (0, 0))
    row = lambda shape: pl.BlockSpec(shape, lambda i: (i, 0))
    return pl.pallas_call(
        _norm_prep_body,
        grid=(N_ENTP // b,),
        in_specs=[row((b, UP)), row((b, UP)), row((b, UP)),
                  full((UP, UP)), full((UP, UP)), full((UP, 1))],
        out_specs=[row((b, UP)), row((b, UP)), row((b, 1)), row((b, 1))],
        out_shape=[jax.ShapeDtypeStruct((N_ENTP, UP), F32),
                   jax.ShapeDtypeStruct((N_ENTP, UP), F32),
                   jax.ShapeDtypeStruct((N_ENTP, 1), F32),
                   jax.ShapeDtypeStruct((N_ENTP, 1), F32)],
    )(p0, p1, hh, wh, wt, ap)


def _final_body(p0, p1, h1, ee, wd, bd, out):
    acc = p0[...] + p1[...]
    denom = acc[:, DCOL:DCOL + 1]
    out1 = (h1[...] * denom + acc) / (denom + 1e-16)
    out[...] = out1 + _dot(ee[...], wd[...]) + bd[...]


def _final(p0, p1, h1, ee, wd, bd):
    b = 2048
    d = ee.shape[1]
    full = lambda shape: pl.BlockSpec(shape, lambda i: (0, 0))
    row = lambda shape: pl.BlockSpec(shape, lambda i: (i, 0))
    return pl.pallas_call(
        _final_body,
        grid=(N_ENTP // b,),
        in_specs=[row((b, UP)), row((b, UP)), row((b, UP)),
                  row((b, d)), full((d, UP)), full((1, UP))],
        out_specs=row((b, UP)),
        out_shape=jax.ShapeDtypeStruct((N_ENTP, UP), F32),
    )(p0, p1, h1, ee, wd, bd)


def _edge_body(hi, ri, ti, sh, sr, st, tt, rr, outp,
               hv, rv, tv, shv, srv, stv,
               rowsR0, rowsT0, rows0, rowsR1, rowsT1, rows1, exb, acc,
               semT0, semR0, semT1, semR1, semS0, semS1):
    c = lax.axis_index("c")
    s = lax.axis_index("s")
    wid = c * NS + s

    zv = jnp.zeros((L,), F32)
    for j in range(K):
        for k2 in range(UP // L):
            rows0[j, pl.ds(k2 * L, L)] = zv
            rows1[j, pl.ds(k2 * L, L)] = zv
    r0 = s * RB
    for i in range(RB // K):
        pltpu.sync_copy(rows0, acc.at[pl.ds(r0 + i * K, K)])

    pltpu.sync_copy(hi.at[wid], hv)
    pltpu.sync_copy(ri.at[wid], rv)
    pltpu.sync_copy(ti.at[wid], tv)
    pltpu.sync_copy(sh, shv)
    pltpu.sync_copy(sr, srv)
    pltpu.sync_copy(st, stv)
    plsc.subcore_barrier()

    def exs(b):
        for i in range(K // L):
            sl = pl.ds(i * L, L)
            lg = (plsc.load_gather(shv, [hv[b, sl]])
                  + plsc.load_gather(srv, [rv[b, sl]])
                  + plsc.load_gather(stv, [tv[b, sl]]))
            lg = jnp.where(lg >= 0.0, lg, lg * 0.2)
            exb[sl] = jnp.exp(lg)

    def scale(rowsR, rowsT, rows):
        def igrp(i, carry):
            ex16 = exb[pl.ds(i * L, L)]
            for j2 in range(L):
                j = i * L + j2
                e = ex16[j2]
                for k2 in range(UP // L):
                    slc = pl.ds(k2 * L, L)
                    rows[j, slc] = (rowsR[j, slc] + rowsT[j, slc]) * e
            return carry
        lax.fori_loop(0, K // L, igrp, 0)

    def gather(b, rowsT, rowsR, semT, semR):
        pltpu.async_copy(tt.at[tv.at[b]], rowsT, semT)
        pltpu.async_copy(rr.at[rv.at[b]], rowsR, semR)

    def wait_gather(rowsT, rowsR, semT, semR):
        pltpu.make_async_copy(tt.at[pl.ds(0, K)], rowsT, semT).wait()
        pltpu.make_async_copy(rr.at[pl.ds(0, K)], rowsR, semR).wait()

    def wait_scatter(rows, semS):
        pltpu.make_async_copy(tt.at[pl.ds(0, K)], rows, semS).wait()

    def half(b, rowsT, rowsR, rows, semT, semR, semS):
        exs(b)
        wait_gather(rowsT, rowsR, semT, semR)
        wait_scatter(rows, semS)
        scale(rowsR, rowsT, rows)
        pltpu.async_copy(rows, acc.at[hv.at[b]], semS, add=True)

    gather(0, rowsT0, rowsR0, semT0, semR0)
    pltpu.async_copy(rows0, acc.at[hv.at[0]], semS0, add=True)
    pltpu.async_copy(rows1, acc.at[hv.at[0]], semS1, add=True)

    def pair(p, carry):
        b0 = 2 * p
        gather(b0 + 1, rowsT1, rowsR1, semT1, semR1)
        half(b0, rowsT0, rowsR0, rows0, semT0, semR0, semS0)
        gather(b0 + 2, rowsT0, rowsR0, semT0, semR0)
        half(b0 + 1, rowsT1, rowsR1, rows1, semT1, semR1, semS1)
        return carry
    lax.fori_loop(0, (NB - 1) // 2, pair, 0)

    half(NB - 1, rowsT0, rowsR0, rows0, semT0, semR0, semS0)
    wait_scatter(rows0, semS0)
    wait_scatter(rows1, semS1)

    plsc.subcore_barrier()
    pltpu.sync_copy(acc.at[pl.ds(r0, RB)], outp.at[c, pl.ds(r0, RB)])


_edge_pass = pl.kernel(
    _edge_body,
    out_type=jax.ShapeDtypeStruct((NC, N_ENTP, UP), F32),
    mesh=plsc.VectorSubcoreMesh(core_axis_name="c", subcore_axis_name="s",
                                num_cores=NC, num_subcores=NS),
    compiler_params=pltpu.CompilerParams(needs_layout_passes=False,
                                         use_tc_tiling_on_sc=False),
    scratch_types=[
        pltpu.VMEM((NB, K), jnp.int32),
        pltpu.VMEM((NB, K), jnp.int32),
        pltpu.VMEM((NB, K), jnp.int32),
        pltpu.VMEM((N_ENTP,), F32),
        pltpu.VMEM((N_RELP,), F32),
        pltpu.VMEM((N_ENTP,), F32),
        pltpu.VMEM((K, UP), F32),
        pltpu.VMEM((K, UP), F32),
        pltpu.VMEM((K, UP), F32),
        pltpu.VMEM((K, UP), F32),
        pltpu.VMEM((K, UP), F32),
        pltpu.VMEM((K, UP), F32),
        pltpu.VMEM((K,), F32),
        pltpu.VMEM_SHARED((N_ENTP, UP), F32),
        pltpu.SemaphoreType.DMA,
        pltpu.SemaphoreType.DMA,
        pltpu.SemaphoreType.DMA,
        pltpu.SemaphoreType.DMA,
        pltpu.SemaphoreType.DMA,
        pltpu.SemaphoreType.DMA,
    ],
)


def kernel(h_index, r_index, t_index, E_entity, E_relation,
           W0, a0, Wr0, W1, a1, Wr1, Wd, bd):
    de, dr, u0, u1 = 128, 128, 50, 50

    padc = lambda w: jnp.pad(w, ((0, 0), (0, UP - w.shape[1])))
    padrc = lambda w: jnp.pad(w, ((0, UP - w.shape[0]), (0, UP - w.shape[1])))

    w0h = padc(W0[:de])
    w0r = padc(W0[de:de + dr])
    w0t = padc(W0[de + dr:])
    a0p = jnp.pad(a0, ((0, UP - u0), (0, 0)))
    wr0p = padc(Wr0)
    w1h = padrc(W1[:u0])
    w1r = padrc(W1[u0:2 * u0])
    w1t = padrc(W1[2 * u0:])
    a1p = jnp.pad(a1, ((0, UP - u1), (0, 0)))
    wr1p = padrc(Wr1)
    wdp = padc(Wd)
    bdp = jnp.pad(bd, (0, UP - u1)).reshape(1, UP)
    erp = jnp.pad(E_relation, ((0, N_RELP - E_relation.shape[0]), (0, 0)))
    eep = jnp.pad(E_entity, ((0, N_ENTP - N_ENT), (0, 0)))

    hi = h_index.reshape(NW, NB, K)
    ri = r_index.reshape(NW, NB, K)
    ti = t_index.reshape(NW, NB, K)

    hh0, tt0, sh0, st0 = _ent_prep(eep, w0h, w0t, a0p)
    rr0, sr0, r1, sr1, reloutp = _rel_prep(erp, w0r, a0p, wr0p, w1r, a1p, wr1p)

    part0 = _edge_pass(hi, ri, ti,
                       sh0.reshape(-1), sr0.reshape(-1), st0.reshape(-1),
                       tt0, rr0)

    h1, t1, sh1, st1 = _norm_prep(part0[0], part0[1], hh0, w1h, w1t, a1p)

    part1 = _edge_pass(hi, ri, ti,
                       sh1.reshape(-1), sr1.reshape(-1), st1.reshape(-1),
                       t1, r1)

    entp = _final(part1[0], part1[1], h1, eep, wdp, bdp)

    return entp[:N_ENT, :u1], reloutp[:E_relation.shape[0], :u1]


# speedup vs baseline: 1.1761x; 1.1761x over previous
import functools

import jax
import jax.numpy as jnp
from jax import lax
from jax.experimental import pallas as pl
from jax.experimental.pallas import tpu as pltpu
from jax.experimental.pallas import tpu_sc as plsc

F32 = jnp.float32
NC, NS, L = 2, 16, 16
NW = NC * NS
N_ENT = 10000
N_ENTP = 10240
N_RELP = 512
UP = 64
DCOL = 50
K = 80
EPT = 320000 // NW
NB = EPT // K
RB = N_ENTP // NS
ZR = 128


def _dot(x, y):
    return jnp.dot(x, y, preferred_element_type=F32)


def _ent_prep_body(ee, wh, wt, ap, hh, tt, sh, st):
    e = ee[...]
    h = _dot(e, wh[...])
    t = _dot(e, wt[...])
    hh[...] = h
    tt[...] = t
    sh[...] = _dot(h, ap[...])
    st[...] = _dot(t, ap[...])


def _ent_prep(ee, wh, wt, ap):
    n, d = ee.shape
    b = 2048
    full = lambda shape: pl.BlockSpec(shape, lambda i: (0, 0))
    row = lambda shape: pl.BlockSpec(shape, lambda i: (i, 0))
    return pl.pallas_call(
        _ent_prep_body,
        grid=(n // b,),
        in_specs=[row((b, d)), full((d, UP)), full((d, UP)), full((UP, 1))],
        out_specs=[row((b, UP)), row((b, UP)), row((b, 1)), row((b, 1))],
        out_shape=[jax.ShapeDtypeStruct((n, UP), F32),
                   jax.ShapeDtypeStruct((n, UP), F32),
                   jax.ShapeDtypeStruct((n, 1), F32),
                   jax.ShapeDtypeStruct((n, 1), F32)],
    )(ee, wh, wt, ap)


def _rel_prep_body(er, w0r, a0p, wr0, w1r, a1p, wr1,
                   rr0, sr0, r1, sr1, relout):
    e = er[...]
    rr = _dot(e, w0r[...])
    rr0[...] = rr
    sr0[...] = _dot(rr, a0p[...])
    rel1 = jnp.maximum(_dot(e, wr0[...]), 0.0)
    r1v = _dot(rel1, w1r[...])
    r1[...] = r1v
    sr1[...] = _dot(r1v, a1p[...])
    relout[...] = _dot(rel1, wr1[...])


def _rel_prep(erp, w0r, a0p, wr0, w1r, a1p, wr1):
    return pl.pallas_call(
        _rel_prep_body,
        out_shape=[jax.ShapeDtypeStruct((N_RELP, UP), F32),
                   jax.ShapeDtypeStruct((N_RELP, 1), F32),
                   jax.ShapeDtypeStruct((N_RELP, UP), F32),
                   jax.ShapeDtypeStruct((N_RELP, 1), F32),
                   jax.ShapeDtypeStruct((N_RELP, UP), F32)],
    )(erp, w0r, a0p, wr0, w1r, a1p, wr1)


def _norm_prep_body(p0, p1, hh, wh, wt, ap, h1, t1, sh, st):
    acc = p0[...] + p1[...]
    denom = acc[:, DCOL:DCOL + 1]
    out0 = (hh[...] * denom + acc) / (denom + 1e-16)
    h = _dot(out0, wh[...])
    t = _dot(out0, wt[...])
    h1[...] = h
    t1[...] = t
    sh[...] = _dot(h, ap[...])
    st[...] = _dot(t, ap[...])


def _norm_prep(p0, p1, hh, wh, wt, ap):
    b = 2048
    full = lambda shape: pl.BlockSpec(shape, lambda i: (0, 0))
    row = lambda shape: pl.BlockSpec(shape, lambda i: (i, 0))
    return pl.pallas_call(
        _norm_prep_body,
        grid=(N_ENTP // b,),
        in_specs=[row((b, UP)), row((b, UP)), row((b, UP)),
                  full((UP, UP)), full((UP, UP)), full((UP, 1))],
        out_specs=[row((b, UP)), row((b, UP)), row((b, 1)), row((b, 1))],
        out_shape=[jax.ShapeDtypeStruct((N_ENTP, UP), F32),
                   jax.ShapeDtypeStruct((N_ENTP, UP), F32),
                   jax.ShapeDtypeStruct((N_ENTP, 1), F32),
                   jax.ShapeDtypeStruct((N_ENTP, 1), F32)],
    )(p0, p1, hh, wh, wt, ap)


def _final_body(p0, p1, h1, ee, wd, bd, out):
    acc = p0[...] + p1[...]
    denom = acc[:, DCOL:DCOL + 1]
    out1 = (h1[...] * denom + acc) / (denom + 1e-16)
    out[...] = out1 + _dot(ee[...], wd[...]) + bd[...]


def _final(p0, p1, h1, ee, wd, bd):
    b = 2048
    d = ee.shape[1]
    full = lambda shape: pl.BlockSpec(shape, lambda i: (0, 0))
    row = lambda shape: pl.BlockSpec(shape, lambda i: (i, 0))
    return pl.pallas_call(
        _final_body,
        grid=(N_ENTP // b,),
        in_specs=[row((b, UP)), row((b, UP)), row((b, UP)),
                  row((b, d)), full((d, UP)), full((1, UP))],
        out_specs=row((b, UP)),
        out_shape=jax.ShapeDtypeStruct((N_ENTP, UP), F32),
    )(p0, p1, h1, ee, wd, bd)


def _edge_body(hi, ri, ti, sh, sr, st, tt, rr, outp,
               hv, rv, tv, shv, srv, stv,
               rowsR0, rowsT0, rows0, rowsR1, rowsT1, rows1, exb, acc, rrs,
               semT0, semR0, semT1, semR1, semS0, semS1):
    c = lax.axis_index("c")
    s = lax.axis_index("s")
    wid = c * NS + s

    zv = jnp.zeros((L,), F32)
    for j in range(K):
        for k2 in range(UP // L):
            rows0[j, pl.ds(k2 * L, L)] = zv
            rows1[j, pl.ds(k2 * L, L)] = zv
    r0 = s * RB
    for i in range(RB // K):
        pltpu.sync_copy(rows0, acc.at[pl.ds(r0 + i * K, K)])

    pltpu.sync_copy(hi.at[wid], hv)
    pltpu.sync_copy(ri.at[wid], rv)
    pltpu.sync_copy(ti.at[wid], tv)
    pltpu.sync_copy(sh, shv)
    pltpu.sync_copy(sr, srv)
    pltpu.sync_copy(st, stv)
    rsl = N_RELP // NS
    pltpu.sync_copy(rr.at[pl.ds(s * rsl, rsl)], rrs.at[pl.ds(s * rsl, rsl)])
    plsc.subcore_barrier()

    lane = lax.iota(jnp.int32, L)
    m01 = lane < (DCOL - 3 * L)
    mD = lane == (DCOL - 3 * L)

    def exs(b):
        for i in range(K // L):
            sl = pl.ds(i * L, L)
            lg = (plsc.load_gather(shv, [hv[b, sl]])
                  + plsc.load_gather(srv, [rv[b, sl]])
                  + plsc.load_gather(stv, [tv[b, sl]]))
            lg = jnp.where(lg >= 0.0, lg, lg * 0.2)
            exb[sl] = jnp.exp(lg)

    def scale(rowsR, rowsT, rows):
        def igrp(i, carry):
            ex16 = exb[pl.ds(i * L, L)]
            for j2 in range(L):
                j = i * L + j2
                e = ex16[j2]
                for k2 in range(3):
                    slc = pl.ds(k2 * L, L)
                    rows[j, slc] = (rowsR[j, slc] + rowsT[j, slc]) * e
                slc = pl.ds(3 * L, L)
                v = (rowsR[j, slc] + rowsT[j, slc]) * e
                rows[j, slc] = jnp.where(m01, v, jnp.where(mD, e, 0.0))
            return carry
        lax.fori_loop(0, K // L, igrp, 0)

    def gather(b, rowsT, rowsR, semT, semR):
        pltpu.async_copy(tt.at[tv.at[b]], rowsT, semT)
        pltpu.async_copy(rrs.at[rv.at[b]], rowsR, semR)

    def wait_gather(rowsT, rowsR, semT, semR):
        pltpu.make_async_copy(tt.at[pl.ds(0, K)], rowsT, semT).wait()
        pltpu.make_async_copy(rrs.at[pl.ds(0, K)], rowsR, semR).wait()

    def wait_scatter(rows, semS):
        pltpu.make_async_copy(tt.at[pl.ds(0, K)], rows, semS).wait()

    def half(b, rowsT, rowsR, rows, semT, semR, semS):
        exs(b)
        wait_gather(rowsT, rowsR, semT, semR)
        wait_scatter(rows, semS)
        scale(rowsR, rowsT, rows)
        pltpu.async_copy(rows, acc.at[hv.at[b]], semS, add=True)

    gather(0, rowsT0, rowsR0, semT0, semR0)
    pltpu.async_copy(rows0, acc.at[hv.at[0]], semS0, add=True)
    pltpu.async_copy(rows1, acc.at[hv.at[0]], semS1, add=True)

    def pair(p, carry):
        b0 = 2 * p
        gather(b0 + 1, rowsT1, rowsR1, semT1, semR1)
        half(b0, rowsT0, rowsR0, rows0, semT0, semR0, semS0)
        gather(b0 + 2, rowsT0, rowsR0, semT0, semR0)
        half(b0 + 1, rowsT1, rowsR1, rows1, semT1, semR1, semS1)
        return carry
    lax.fori_loop(0, (NB - 1) // 2, pair, 0)

    half(NB - 1, rowsT0, rowsR0, rows0, semT0, semR0, semS0)
    wait_scatter(rows0, semS0)
    wait_scatter(rows1, semS1)

    plsc.subcore_barrier()
    pltpu.sync_copy(acc.at[pl.ds(r0, RB)], outp.at[c, pl.ds(r0, RB)])


_edge_pass = pl.kernel(
    _edge_body,
    out_type=jax.ShapeDtypeStruct((NC, N_ENTP, UP), F32),
    mesh=plsc.VectorSubcoreMesh(core_axis_name="c", subcore_axis_name="s",
                                num_cores=NC, num_subcores=NS),
    compiler_params=pltpu.CompilerParams(needs_layout_passes=False,
                                         use_tc_tiling_on_sc=False),
    scratch_types=[
        pltpu.VMEM((NB, K), jnp.int32),
        pltpu.VMEM((NB, K), jnp.int32),
        pltpu.VMEM((NB, K), jnp.int32),
        pltpu.VMEM((N_ENTP,), F32),
        pltpu.VMEM((N_RELP,), F32),
        pltpu.VMEM((N_ENTP,), F32),
        pltpu.VMEM((K, UP), F32),
        pltpu.VMEM((K, UP), F32),
        pltpu.VMEM((K, UP), F32),
        pltpu.VMEM((K, UP), F32),
        pltpu.VMEM((K, UP), F32),
        pltpu.VMEM((K, UP), F32),
        pltpu.VMEM((K,), F32),
        pltpu.VMEM_SHARED((N_ENTP, UP), F32),
        pltpu.VMEM_SHARED((N_RELP, UP), F32),
        pltpu.SemaphoreType.DMA,
        pltpu.SemaphoreType.DMA,
        pltpu.SemaphoreType.DMA,
        pltpu.SemaphoreType.DMA,
        pltpu.SemaphoreType.DMA,
        pltpu.SemaphoreType.DMA,
    ],
)


def kernel(h_index, r_index, t_index, E_entity, E_relation,
           W0, a0, Wr0, W1, a1, Wr1, Wd, bd):
    de, dr, u0, u1 = 128, 128, 50, 50

    padc = lambda w: jnp.pad(w, ((0, 0), (0, UP - w.shape[1])))
    padrc = lambda w: jnp.pad(w, ((0, UP - w.shape[0]), (0, UP - w.shape[1])))

    w0h = padc(W0[:de])
    w0r = padc(W0[de:de + dr])
    w0t = padc(W0[de + dr:])
    a0p = jnp.pad(a0, ((0, UP - u0), (0, 0)))
    wr0p = padc(Wr0)
    w1h = padrc(W1[:u0])
    w1r = padrc(W1[u0:2 * u0])
    w1t = padrc(W1[2 * u0:])
    a1p = jnp.pad(a1, ((0, UP - u1), (0, 0)))
    wr1p = padrc(Wr1)
    wdp = padc(Wd)
    bdp = jnp.pad(bd, (0, UP - u1)).reshape(1, UP)
    erp = jnp.pad(E_relation, ((0, N_RELP - E_relation.shape[0]), (0, 0)))
    eep = jnp.pad(E_entity, ((0, N_ENTP - N_ENT), (0, 0)))

    hi = h_index.reshape(NW, NB, K)
    ri = r_index.reshape(NW, NB, K)
    ti = t_index.reshape(NW, NB, K)

    hh0, tt0, sh0, st0 = _ent_prep(eep, w0h, w0t, a0p)
    rr0, sr0, r1, sr1, reloutp = _rel_prep(erp, w0r, a0p, wr0p, w1r, a1p, wr1p)

    part0 = _edge_pass(hi, ri, ti,
                       sh0.reshape(-1), sr0.reshape(-1), st0.reshape(-1),
                       tt0, rr0)

    h1, t1, sh1, st1 = _norm_prep(part0[0], part0[1], hh0, w1h, w1t, a1p)

    part1 = _edge_pass(hi, ri, ti,
                       sh1.reshape(-1), sr1.reshape(-1), st1.reshape(-1),
                       t1, r1)

    entp = _final(part1[0], part1[1], h1, eep, wdp, bdp)

    return entp[:N_ENT, :u1], reloutp[:E_relation.shape[0], :u1]
